# Initial kernel scaffold; baseline (speedup 1.0000x reference)
import jax
import jax.numpy as jnp
from jax.experimental import pallas as pl


def kernel(hidden_states, attention_msk, tree_lens, edge_index, Wf, bf, W1, b1, W2, b2, Wfc, bfc):
    raise NotImplementedError("write your pallas kernel here")


# speedup vs baseline: 6.0742x; 6.0742x over previous
import jax
import jax.numpy as jnp
from jax.experimental import pallas as pl
from jax.experimental.pallas import tpu as pltpu

_B = 16
_T = 64
_L = 64
_H = 768
_E = 63


def _gcn_pool_kernel(hs_ref, msk_ref, edge_ref, wf_ref, bf_ref,
                     w1_ref, b1_ref, w2_ref, b2_ref, wfc_ref, bfc_ref,
                     out_ref):
    f32 = jnp.float32
    hs = hs_ref[0]
    m = msk_ref[0]

    col_tweet = jax.lax.broadcasted_iota(jnp.int32, (_T, _T * _L), 1) // _L
    row_tweet = jax.lax.broadcasted_iota(jnp.int32, (_T, _T * _L), 0)
    pool = jnp.where(col_tweet == row_tweet,
                     jnp.broadcast_to(m, (_T, _T * _L)), 0.0)
    sums = jnp.dot(pool, hs, preferred_element_type=f32)
    cnts = jnp.sum(pool, axis=1, keepdims=True)
    nodes = sums / jnp.maximum(cnts, 1e-9)

    src = edge_ref[0, 0:1, :]
    dst = edge_ref[0, 1:2, :]
    node_ids = jax.lax.broadcasted_iota(jnp.int32, (_T, _E), 0)
    oh_srcT = (node_ids == src).astype(f32)
    oh_dstT = (node_ids == dst).astype(f32)
    s_nodes = jax.lax.dot_general(wf_ref[...], nodes,
                                  (((1,), (1,)), ((), ())),
                                  preferred_element_type=f32)
    sc_e = jnp.dot(s_nodes[0:1, :], oh_dstT, preferred_element_type=f32)
    sp_e = jnp.dot(s_nodes[1:2, :], oh_srcT, preferred_element_type=f32)
    ew = jax.nn.sigmoid(sc_e + sp_e + bf_ref[0, 0])

    a_w = jax.lax.dot_general(oh_dstT * ew, oh_srcT,
                              (((1,), (1,)), ((), ())),
                              preferred_element_type=f32)
    eye = (jax.lax.broadcasted_iota(jnp.int32, (_T, _T), 0)
           == jax.lax.broadcasted_iota(jnp.int32, (_T, _T), 1)).astype(f32)
    a_sl = a_w + eye
    deg = jnp.sum(a_sl, axis=1, keepdims=True)
    dinv = jnp.where(deg > 0,
                     jax.lax.rsqrt(jnp.maximum(deg, 1e-12)), 0.0)

    h1 = jnp.dot(nodes, w1_ref[...], preferred_element_type=f32)
    x1 = jnp.maximum(dinv * jnp.dot(a_sl, dinv * h1,
                                    preferred_element_type=f32)
                     + b1_ref[...], 0.0)
    h2 = jnp.dot(x1, w2_ref[...], preferred_element_type=f32)
    x2 = jnp.maximum(dinv * jnp.dot(a_sl, dinv * h2,
                                    preferred_element_type=f32)
                     + b2_ref[...], 0.0)

    pooled = jnp.sum(x2, axis=0, keepdims=True) * (1.0 / _T)
    out_ref[0] = jnp.tanh(jnp.dot(pooled, wfc_ref[...],
                                  preferred_element_type=f32)
                          + bfc_ref[...])


def kernel(hidden_states, attention_msk, tree_lens, edge_index,
           Wf, bf, W1, b1, W2, b2, Wfc, bfc):
    del tree_lens
    msk3 = attention_msk.reshape(_B, 1, _T * _L)
    wf2 = Wf.reshape(2, _H)
    bf2 = bf.reshape(1, 1)
    b1r = b1.reshape(1, _H)
    b2r = b2.reshape(1, _H)
    bfcr = bfc.reshape(1, _H)

    return pl.pallas_call(
        _gcn_pool_kernel,
        grid=(_B,),
        in_specs=[
            pl.BlockSpec((1, _T * _L, _H), lambda i: (i, 0, 0)),
            pl.BlockSpec((1, 1, _T * _L), lambda i: (i, 0, 0)),
            pl.BlockSpec((1, 2, _E), lambda i: (i, 0, 0)),
            pl.BlockSpec((2, _H), lambda i: (0, 0)),
            pl.BlockSpec((1, 1), lambda i: (0, 0)),
            pl.BlockSpec((_H, _H), lambda i: (0, 0)),
            pl.BlockSpec((1, _H), lambda i: (0, 0)),
            pl.BlockSpec((_H, _H), lambda i: (0, 0)),
            pl.BlockSpec((1, _H), lambda i: (0, 0)),
            pl.BlockSpec((_H, _H), lambda i: (0, 0)),
            pl.BlockSpec((1, _H), lambda i: (0, 0)),
        ],
        out_specs=pl.BlockSpec((1, 1, _H), lambda i: (i, 0, 0)),
        out_shape=jax.ShapeDtypeStruct((_B, 1, _H), jnp.float32),
        compiler_params=pltpu.CompilerParams(
            dimension_semantics=("arbitrary",)),
    )(hidden_states, msk3, edge_index, wf2, bf2,
      W1, b1r, W2, b2r, Wfc, bfcr).reshape(_B, _H)
